# Initial kernel scaffold; baseline (speedup 1.0000x reference)
import jax
import jax.numpy as jnp
from jax.experimental import pallas as pl


def kernel(x, edge_index, W_s, b_s, W_mid, b_mid, W_e, b_e, Wd_s, bd_s, Wd_mid, bd_mid, Wd_e, bd_e):
    raise NotImplementedError("write your pallas kernel here")


# speedup vs baseline: 4.2242x; 4.2242x over previous
import functools

import jax
import jax.numpy as jnp
from jax import lax
from jax.experimental import pallas as pl
from jax.experimental.pallas import tpu as pltpu
from jax.experimental.pallas import tpu_sc as plsc

N = 50000
E = 800000
NC = 2
NS = 16
CH = 128
NQ = 8
QW = 8

NACC = 50048
RPT = NACC // NS

NSTEP = 392
EPT = NSTEP * CH
NSTEP_D = 200
EPT_D = NSTEP_D * CH

BR = 1024
GRID = (NACC + BR - 1) // BR


def _mesh():
    return plsc.VectorSubcoreMesh(
        core_axis_name="c", subcore_axis_name="s",
        num_cores=NC, num_subcores=NS)


NROW_D = EPT_D // 16


@functools.cache
def _build_sc_deg():
    return pl.kernel(
        _sc_deg,
        out_type=jax.ShapeDtypeStruct((NC * NS * NACC,), jnp.float32),
        mesh=_mesh(),
        scratch_types=[
            pltpu.VMEM((NROW_D, 16), jnp.int32),
            pltpu.VMEM((NACC,), jnp.float32),
        ],
        compiler_params=pltpu.CompilerParams(
            use_tc_tiling_on_sc=False, needs_layout_passes=False),
    )


def _sc_deg(dst_hbm, zeros_hbm, out_hbm, idx_v, hist):
    c = lax.axis_index("c")
    s = lax.axis_index("s")
    w = c * NS + s
    pltpu.sync_copy(zeros_hbm, hist)
    pltpu.sync_copy(dst_hbm.at[pl.ds(w * NROW_D, NROW_D)], idx_v)
    ones16 = jnp.ones((16,), jnp.float32)

    def body(j, carry):
        plsc.addupdate_scatter(hist, [idx_v[j]], ones16)
        return carry

    lax.fori_loop(0, NROW_D, body, 0)
    pltpu.sync_copy(hist, out_hbm.at[pl.ds(w * NACC, NACC)])


@functools.cache
def _build_sc_edge():
    return pl.kernel(
        _sc_edge,
        out_type=jax.ShapeDtypeStruct((NQ * NACC, QW), jnp.float32),
        mesh=_mesh(),
        scratch_types=[
            pltpu.VMEM((NSTEP, CH), jnp.int32),
            pltpu.VMEM((NSTEP, CH), jnp.int32),
            pltpu.VMEM((CH, QW), jnp.float32),
            pltpu.VMEM_SHARED((NACC, QW), jnp.float32),
            pltpu.SemaphoreType.DMA,
        ],
        compiler_params=pltpu.CompilerParams(use_tc_tiling_on_sc=False),
    )


def _sc_edge(g_hbm, src_hbm, dst_hbm, zeros_hbm, out_hbm,
             srcv, dstv, rows, acc, sem):
    c = lax.axis_index("c")
    s = lax.axis_index("s")
    pltpu.sync_copy(dst_hbm.at[pl.ds(s * NSTEP, NSTEP)], dstv)

    def quarter_pass(qi, carry):
        q = c * 4 + qi
        pltpu.sync_copy(zeros_hbm.at[pl.ds(s * RPT, RPT)],
                        acc.at[pl.ds(s * RPT, RPT)])
        pltpu.sync_copy(src_hbm.at[pl.ds((q * NS + s) * NSTEP, NSTEP)], srcv)
        plsc.subcore_barrier()

        def body(j, inner):
            pltpu.async_copy(g_hbm.at[srcv.at[j]], rows, sem).wait()
            pltpu.sync_copy(rows, acc.at[dstv.at[j]], add=True)
            return inner

        lax.fori_loop(0, NSTEP, body, 0)
        plsc.subcore_barrier()
        pltpu.sync_copy(acc.at[pl.ds(s * RPT, RPT)],
                        out_hbm.at[pl.ds(q * NACC + s * RPT, RPT)])
        plsc.subcore_barrier()
        return carry

    lax.fori_loop(0, 4, quarter_pass, 0)


def _tc_pre(degp, x, w_s):
    def body(degp_ref, x_ref, w_ref, dinv_ref, g_ref):
        deg = jnp.sum(degp_ref[...], axis=0) + 1.0
        dinv = lax.rsqrt(deg)[:, None]
        dinv_ref[...] = dinv
        xb = x_ref[...]
        w = w_ref[...]
        h = (xb[:, 0:1] * w[0:1, :] + xb[:, 1:2] * w[1:2, :]
             + xb[:, 2:3] * w[2:3, :])
        g = dinv * h
        for q in range(NQ):
            g_ref[q] = g[:, q * QW:(q + 1) * QW]

    return pl.pallas_call(
        body,
        grid=(GRID,),
        in_specs=[
            pl.BlockSpec((NC * NS, BR), lambda i: (0, i)),
            pl.BlockSpec((BR, 3), lambda i: (i, 0)),
            pl.BlockSpec((3, 64), lambda i: (0, 0)),
        ],
        out_specs=[
            pl.BlockSpec((BR, 1), lambda i: (i, 0)),
            pl.BlockSpec((NQ, BR, QW), lambda i: (0, i, 0)),
        ],
        out_shape=[
            jax.ShapeDtypeStruct((NACC, 1), jnp.float32),
            jax.ShapeDtypeStruct((NQ, NACC, QW), jnp.float32),
        ],
    )(degp, x, w_s)


def _tc_mid(agg, g, dinv, b4, w):
    def body(agg_ref, g_ref, dinv_ref, b_ref, w_ref, out_ref):
        dinv = dinv_ref[...]
        h = jnp.maximum(dinv[None] * (agg_ref[...] + g_ref[...]) + b_ref[...],
                        0.0)
        wm = w_ref[...]
        hw = sum(jnp.dot(h[q], wm[q * QW:(q + 1) * QW, :],
                         preferred_element_type=jnp.float32)
                 for q in range(NQ))
        gn = dinv * hw
        for q in range(NQ):
            out_ref[q] = gn[:, q * QW:(q + 1) * QW]

    return pl.pallas_call(
        body,
        grid=(GRID,),
        in_specs=[
            pl.BlockSpec((NQ, BR, QW), lambda i: (0, i, 0)),
            pl.BlockSpec((NQ, BR, QW), lambda i: (0, i, 0)),
            pl.BlockSpec((BR, 1), lambda i: (i, 0)),
            pl.BlockSpec((NQ, 1, QW), lambda i: (0, 0, 0)),
            pl.BlockSpec((64, 64), lambda i: (0, 0)),
        ],
        out_specs=pl.BlockSpec((NQ, BR, QW), lambda i: (0, i, 0)),
        out_shape=jax.ShapeDtypeStruct((NQ, NACC, QW), jnp.float32),
    )(agg, g, dinv, b4, w)


def _tc_final(agg, g, dinv, b4, x, wd_s, bd_s, wd_mid, bd_mid, wd_e, bd_e):
    def body(agg_ref, g_ref, dinv_ref, b_ref, x_ref, wds_ref, bds_ref,
             wdm_ref, bdm_ref, wde_ref, bde_ref, y_ref):
        dinv = dinv_ref[...]
        h4 = jnp.maximum(dinv[None] * (agg_ref[...] + g_ref[...]) + b_ref[...],
                         0.0)
        wds = wds_ref[...]
        h = sum(jnp.dot(h4[q], wds[q * QW:(q + 1) * QW, :],
                        preferred_element_type=jnp.float32)
                for q in range(NQ))
        h = jnp.maximum(h + bds_ref[...], 0.0)
        wdm = wdm_ref[...]
        h = jnp.maximum(
            jnp.dot(h, wdm, preferred_element_type=jnp.float32) + bdm_ref[...],
            0.0)
        h = jnp.maximum(
            jnp.dot(h, wdm, preferred_element_type=jnp.float32) + bdm_ref[...],
            0.0)
        y = (jnp.dot(h, wde_ref[...], preferred_element_type=jnp.float32)
             + bde_ref[...] + x_ref[...])
        y_ref[...] = y

    return pl.pallas_call(
        body,
        grid=(GRID,),
        in_specs=[
            pl.BlockSpec((NQ, BR, QW), lambda i: (0, i, 0)),
            pl.BlockSpec((NQ, BR, QW), lambda i: (0, i, 0)),
            pl.BlockSpec((BR, 1), lambda i: (i, 0)),
            pl.BlockSpec((NQ, 1, QW), lambda i: (0, 0, 0)),
            pl.BlockSpec((BR, 3), lambda i: (i, 0)),
            pl.BlockSpec((64, 64), lambda i: (0, 0)),
            pl.BlockSpec((1, 64), lambda i: (0, 0)),
            pl.BlockSpec((64, 64), lambda i: (0, 0)),
            pl.BlockSpec((1, 64), lambda i: (0, 0)),
            pl.BlockSpec((64, 3), lambda i: (0, 0)),
            pl.BlockSpec((1, 3), lambda i: (0, 0)),
        ],
        out_specs=pl.BlockSpec((BR, 3), lambda i: (i, 0)),
        out_shape=jax.ShapeDtypeStruct((N, 3), jnp.float32),
    )(agg, g, dinv, b4, x, wd_s, bd_s, wd_mid, bd_mid, wd_e, bd_e)


def kernel(x, edge_index, W_s, b_s, W_mid, b_mid, W_e, b_e,
           Wd_s, bd_s, Wd_mid, bd_mid, Wd_e, bd_e):
    src = edge_index[0].astype(jnp.int32)
    dst = edge_index[1].astype(jnp.int32)

    pad_e = NS * EPT - E
    src_p = jnp.concatenate([src, jnp.zeros((pad_e,), jnp.int32)])
    dst_p = jnp.concatenate([dst, jnp.full((pad_e,), N, jnp.int32)])
    src_t = src_p.reshape(NS * NSTEP, CH)
    src4 = jnp.concatenate([src_t + q * NACC for q in range(NQ)], axis=0)
    dst_t = dst_p.reshape(NS * NSTEP, CH)

    pad_d = NC * NS * EPT_D - E
    dst_d = jnp.concatenate([dst, jnp.full((pad_d,), N, jnp.int32)])
    dst_d = dst_d.reshape(NC * NS * NROW_D, 16)

    zeros1 = jnp.zeros((NACC,), jnp.float32)
    zerosq = jnp.zeros((NACC, QW), jnp.float32)

    degp = _build_sc_deg()(dst_d, zeros1).reshape(NC * NS, NACC)
    dinv, g = _tc_pre(degp, x, W_s)

    edge_pass = _build_sc_edge()

    b_stack = jnp.stack([b_s.reshape(NQ, 1, QW), b_mid.reshape(NQ, 1, QW),
                         b_mid.reshape(NQ, 1, QW), b_e.reshape(NQ, 1, QW)])
    w_stack = jnp.stack([W_mid, W_mid, W_e, W_e])

    def layer(i, carry):
        gc, _, _ = carry
        agg = edge_pass(gc.reshape(NQ * NACC, QW), src4, dst_t,
                        zerosq).reshape(NQ, NACC, QW)
        bb = lax.dynamic_index_in_dim(b_stack, i, keepdims=False)
        wn = lax.dynamic_index_in_dim(w_stack, i, keepdims=False)
        return _tc_mid(agg, gc, dinv, bb, wn), agg, gc

    _, agg, g = lax.fori_loop(0, 4, layer, (g, g, g))
    return _tc_final(agg, g, dinv, b_e.reshape(NQ, 1, QW), x,
                     Wd_s, bd_s.reshape(1, 64), Wd_mid, bd_mid.reshape(1, 64),
                     Wd_e, bd_e.reshape(1, 3))
